# Initial kernel scaffold; baseline (speedup 1.0000x reference)
import jax
import jax.numpy as jnp
from jax.experimental import pallas as pl


def kernel(y_score, y_partial, W1, b1, W2, b2):
    raise NotImplementedError("write your pallas kernel here")


# speedup vs baseline: 680.7987x; 680.7987x over previous
import functools

import jax
import jax.numpy as jnp
from jax import lax
from jax.experimental import pallas as pl
from jax.experimental.pallas import tpu as pltpu
from jax.experimental.pallas import tpu_sc as plsc

_BINS = 128
_C = 26
_BATCH = 16384
_N = _BATCH * _C
_NC, _NS, _L = 2, 16, 16
_NW = _NC * _NS
_CHUNK = _N // _NW
_STEPS = _CHUNK // _L
_PERIOD = 13
_OUTER = _STEPS // _PERIOD
_FB = _C * _BINS
_HSTRIDE = _FB + 1
_HWORDS = ((_L * _HSTRIDE + 255) // 256) * 256
_TAB = 130
_TABN = _C * _TAB
_BPS = _FB // _NS

_MESH = plsc.VectorSubcoreMesh(core_axis_name="c", subcore_axis_name="s")


def _class_offsets(scale):
    lane = lax.broadcasted_iota(jnp.int32, (_L,), 0)
    offs = []
    for k in range(_PERIOD):
        cv = lane + (_L * k) % _C
        cv = jnp.where(cv >= _C, cv - _C, cv)
        offs.append(cv * scale)
    return offs


@functools.partial(
    pl.kernel,
    out_type=jax.ShapeDtypeStruct((_NC * _FB,), jnp.float32),
    mesh=_MESH,
    compiler_params=pltpu.CompilerParams(needs_layout_passes=False),
    scratch_types=[
        pltpu.VMEM((_CHUNK,), jnp.float32),
        pltpu.VMEM((_CHUNK,), jnp.int32),
        pltpu.VMEM((_HWORDS,), jnp.float32),
        pltpu.VMEM((_FB,), jnp.float32),
        pltpu.VMEM_SHARED((_NS * _FB,), jnp.float32),
        pltpu.VMEM((_NS * _BPS,), jnp.float32),
        pltpu.VMEM((_BPS,), jnp.float32),
    ],
)
def _hist_call(s_hbm, p_hbm, cnt_hbm, s_v, p_v, h_v, r_v, shared, cls_v, o_v):
    cid = lax.axis_index("c")
    sid = lax.axis_index("s")
    wid = cid * _NS + sid
    base = wid * _CHUNK
    pltpu.sync_copy(s_hbm.at[pl.ds(base, _CHUNK)], s_v)
    pltpu.sync_copy(p_hbm.at[pl.ds(base, _CHUNK)], p_v)

    zero = jnp.zeros((_L,), jnp.float32)

    def zbody(i, carry):
        b = i * 256
        for k in range(16):
            h_v[pl.ds(b + k * _L, _L)] = zero
        return carry

    lax.fori_loop(0, _HWORDS // 256, zbody, 0)

    lane = lax.broadcasted_iota(jnp.int32, (_L,), 0)
    lane_off = lane * _HSTRIDE
    coffs = [c + lane_off for c in _class_offsets(_BINS)]

    def mbody(o, carry):
        b0 = o * (_PERIOD * _L)
        for k in range(_PERIOD):
            off = b0 + k * _L
            s16 = s_v[pl.ds(off, _L)]
            p16 = p_v[pl.ds(off, _L)]
            bin_ = jnp.minimum((s16 * 128.0).astype(jnp.int32), _BINS - 1)
            plsc.addupdate_scatter(h_v, [coffs[k] + bin_], p16.astype(jnp.float32))
        return carry

    lax.fori_loop(0, _OUTER, mbody, 0)

    def rbody(j, carry):
        b = j * _L
        acc = h_v[pl.ds(b, _L)]
        for l in range(1, _L):
            acc = acc + h_v[pl.ds(l * _HSTRIDE + b, _L)]
        r_v[pl.ds(b, _L)] = acc
        return carry

    lax.fori_loop(0, _FB // _L, rbody, 0)

    pltpu.sync_copy(r_v, shared.at[pl.ds(sid * _FB, _FB)])
    plsc.subcore_barrier()
    for l in range(_NS):
        pltpu.sync_copy(shared.at[pl.ds(l * _FB + sid * _BPS, _BPS)],
                        cls_v.at[pl.ds(l * _BPS, _BPS)])

    def cbody(k, carry):
        b = k * _L
        acc = cls_v[pl.ds(b, _L)]
        for l in range(1, _NS):
            acc = acc + cls_v[pl.ds(l * _BPS + b, _L)]
        o_v[pl.ds(b, _L)] = acc
        return carry

    lax.fori_loop(0, _BPS // _L, cbody, 0)
    pltpu.sync_copy(o_v, cnt_hbm.at[pl.ds(cid * _FB + sid * _BPS, _BPS)])


def _fit_kernel(cnt_ref, w1_ref, b1_ref, w2_ref, b2_ref, ta_ref, tb_ref):
    cnt = cnt_ref[0] + cnt_ref[1]
    total = jnp.sum(cnt, axis=1, keepdims=True)
    hist = cnt / total
    h = jnp.clip(hist, 1e-6, 1.0 - 1e-6)
    h = jnp.log(h / (1.0 - h))
    h = lax.dot_general(h, w1_ref[...], (((1,), (1,)), ((), ())),
                        precision=lax.Precision.HIGHEST,
                        preferred_element_type=jnp.float32) + b1_ref[...]
    h = jnp.where(h >= 0.0, h, 0.01 * h)
    d = lax.dot_general(h, w2_ref[...], (((1,), (1,)), ((), ())),
                        precision=lax.Precision.HIGHEST,
                        preferred_element_type=jnp.float32) + b2_ref[...]
    mx = jnp.max(d, axis=1, keepdims=True)
    e = jnp.exp(d - mx)
    p = e / jnp.sum(e, axis=1, keepdims=True)
    rr = lax.broadcasted_iota(jnp.int32, (_BINS, _BINS), 0)
    cc = lax.broadcasted_iota(jnp.int32, (_BINS, _BINS), 1)
    tri = (rr <= cc).astype(jnp.float32)
    y = lax.dot_general(p, tri, (((1,), (0,)), ((), ())),
                        precision=lax.Precision.HIGHEST,
                        preferred_element_type=jnp.float32)
    e0 = y - p
    ji = lax.broadcasted_iota(jnp.int32, (1, _BINS), 1)
    j = ji.astype(jnp.float32)
    dxinv = jnp.where(ji == 0, 256.0, 128.0)
    x0 = jnp.where(ji == 0, 0.0, (2.0 * j - 1.0) / 256.0)
    bt = p * dxinv
    at = e0 - bt * x0
    ta_ref[:, 0:_BINS] = at
    tb_ref[:, 0:_BINS] = bt
    y127 = y[:, _BINS - 1:_BINS]
    b128 = (1.0 - y127) * 256.0
    a128 = y127 - b128 * (255.0 / 256.0)
    ta_ref[:, _BINS:_BINS + 1] = a128
    tb_ref[:, _BINS:_BINS + 1] = b128
    zcol = jnp.zeros((_C, 1), jnp.float32)
    ta_ref[:, _BINS + 1:_BINS + 2] = zcol
    tb_ref[:, _BINS + 1:_BINS + 2] = zcol


_fit_call = pl.pallas_call(
    _fit_kernel,
    out_shape=(
        jax.ShapeDtypeStruct((_C, _TAB), jnp.float32),
        jax.ShapeDtypeStruct((_C, _TAB), jnp.float32),
    ),
)


@functools.partial(
    pl.kernel,
    out_type=jax.ShapeDtypeStruct((_N,), jnp.float32),
    mesh=_MESH,
    compiler_params=pltpu.CompilerParams(needs_layout_passes=False),
    scratch_types=[
        pltpu.VMEM((_CHUNK,), jnp.float32),
        pltpu.VMEM((_CHUNK,), jnp.int32),
        pltpu.VMEM((_TABN,), jnp.float32),
        pltpu.VMEM((_TABN,), jnp.float32),
        pltpu.VMEM((_CHUNK,), jnp.float32),
    ],
)
def _interp_call(s_hbm, p_hbm, ta_hbm, tb_hbm, out_hbm, s_v, p_v, ta_v, tb_v, o_v):
    cid = lax.axis_index("c")
    sid = lax.axis_index("s")
    wid = cid * _NS + sid
    base = wid * _CHUNK
    pltpu.sync_copy(s_hbm.at[pl.ds(base, _CHUNK)], s_v)
    pltpu.sync_copy(p_hbm.at[pl.ds(base, _CHUNK)], p_v)
    pltpu.sync_copy(ta_hbm, ta_v)
    pltpu.sync_copy(tb_hbm, tb_v)

    tcoffs = _class_offsets(_TAB)
    ones = jnp.ones((_L,), jnp.float32)

    def mbody(o, carry):
        b0 = o * (_PERIOD * _L)
        for k in range(_PERIOD):
            off = b0 + k * _L
            s16 = s_v[pl.ds(off, _L)]
            p16 = p_v[pl.ds(off, _L)]
            i16 = jnp.minimum((s16 * 128.0 + 0.5).astype(jnp.int32), _BINS)
            idx = tcoffs[k] + i16
            a = plsc.load_gather(ta_v, [idx])
            b = plsc.load_gather(tb_v, [idx])
            w = a + b * s16
            o_v[pl.ds(off, _L)] = jnp.where(p16 == 1, w, ones)
        return carry

    lax.fori_loop(0, _OUTER, mbody, 0)
    pltpu.sync_copy(o_v, out_hbm.at[pl.ds(base, _CHUNK)])


def kernel(y_score, y_partial, W1, b1, W2, b2):
    s_flat = y_score.reshape(_N)
    p_flat = y_partial.astype(jnp.int32).reshape(_N)
    cnt = _hist_call(s_flat, p_flat)
    ta, tb = _fit_call(cnt.reshape(_NC, _C, _BINS), W1,
                       b1.reshape(1, _BINS), W2, b2.reshape(1, _BINS))
    out = _interp_call(s_flat, p_flat, ta.reshape(_TABN), tb.reshape(_TABN))
    return out.reshape(_BATCH, _C)


# speedup vs baseline: 689.4465x; 1.0127x over previous
import functools

import jax
import jax.numpy as jnp
from jax import lax
from jax.experimental import pallas as pl
from jax.experimental.pallas import tpu as pltpu
from jax.experimental.pallas import tpu_sc as plsc

_BINS = 128
_C = 26
_BATCH = 16384
_N = _BATCH * _C
_NC, _NS, _L = 2, 16, 16
_NW = _NC * _NS
_CHUNK = _N // _NW
_STEPS = _CHUNK // _L
_PERIOD = 13
_OUTER = _STEPS // _PERIOD
_FB = _C * _BINS
_HSTRIDE = _FB + 1
_HWORDS = ((_L * _HSTRIDE + 255) // 256) * 256
_BPS = _FB // _NS
_TROWS = 32
_TABN = _TROWS * _BINS

_MESH = plsc.VectorSubcoreMesh(core_axis_name="c", subcore_axis_name="s")


def _class_offsets(scale):
    lane = lax.broadcasted_iota(jnp.int32, (_L,), 0)
    offs = []
    for k in range(_PERIOD):
        cv = lane + (_L * k) % _C
        cv = jnp.where(cv >= _C, cv - _C, cv)
        offs.append(cv * scale)
    return offs


@functools.partial(
    pl.kernel,
    out_type=jax.ShapeDtypeStruct((_NC * _FB,), jnp.float32),
    mesh=_MESH,
    compiler_params=pltpu.CompilerParams(needs_layout_passes=False),
    scratch_types=[
        pltpu.VMEM((_CHUNK,), jnp.float32),
        pltpu.VMEM((_CHUNK,), jnp.int32),
        pltpu.VMEM((_HWORDS,), jnp.float32),
        pltpu.VMEM((_FB,), jnp.float32),
        pltpu.VMEM_SHARED((_NS * _FB,), jnp.float32),
        pltpu.VMEM((_NS * _BPS,), jnp.float32),
        pltpu.VMEM((_BPS,), jnp.float32),
    ],
)
def _hist_call(s_hbm, p_hbm, cnt_hbm, s_v, p_v, h_v, r_v, shared, cls_v, o_v):
    cid = lax.axis_index("c")
    sid = lax.axis_index("s")
    wid = cid * _NS + sid
    base = wid * _CHUNK
    pltpu.sync_copy(s_hbm.at[pl.ds(base, _CHUNK)], s_v)
    pltpu.sync_copy(p_hbm.at[pl.ds(base, _CHUNK)], p_v)

    zero = jnp.zeros((_L,), jnp.float32)

    def zbody(i, carry):
        b = i * 256
        for k in range(16):
            h_v[pl.ds(b + k * _L, _L)] = zero
        return carry

    lax.fori_loop(0, _HWORDS // 256, zbody, 0)

    lane = lax.broadcasted_iota(jnp.int32, (_L,), 0)
    lane_off = lane * _HSTRIDE
    coffs = [c + lane_off for c in _class_offsets(_BINS)]

    def mbody(o, carry):
        b0 = o * (_PERIOD * _L)
        for k in range(_PERIOD):
            off = b0 + k * _L
            s16 = s_v[pl.ds(off, _L)]
            p16 = p_v[pl.ds(off, _L)]
            bin_ = jnp.minimum((s16 * 128.0).astype(jnp.int32), _BINS - 1)
            plsc.addupdate_scatter(h_v, [coffs[k] + bin_], p16.astype(jnp.float32))
        return carry

    lax.fori_loop(0, _OUTER, mbody, 0)

    def rbody(j, carry):
        b = j * _L
        acc = h_v[pl.ds(b, _L)]
        for l in range(1, _L):
            acc = acc + h_v[pl.ds(l * _HSTRIDE + b, _L)]
        r_v[pl.ds(b, _L)] = acc
        return carry

    lax.fori_loop(0, _FB // _L, rbody, 0)

    pltpu.sync_copy(r_v, shared.at[pl.ds(sid * _FB, _FB)])
    plsc.subcore_barrier()
    for l in range(_NS):
        pltpu.sync_copy(shared.at[pl.ds(l * _FB + sid * _BPS, _BPS)],
                        cls_v.at[pl.ds(l * _BPS, _BPS)])

    def cbody(k, carry):
        b = k * _L
        acc = cls_v[pl.ds(b, _L)]
        for l in range(1, _NS):
            acc = acc + cls_v[pl.ds(l * _BPS + b, _L)]
        o_v[pl.ds(b, _L)] = acc
        return carry

    lax.fori_loop(0, _BPS // _L, cbody, 0)
    pltpu.sync_copy(o_v, cnt_hbm.at[pl.ds(cid * _FB + sid * _BPS, _BPS)])


def _fit_kernel(cnt_ref, w1_ref, b1_ref, w2_ref, b2_ref, ta_ref, tb_ref):
    cnt2 = jnp.reshape(cnt_ref[...], (2 * _C, _BINS))
    cnt = cnt2[0:_C] + cnt2[_C:2 * _C]
    total = jnp.sum(cnt, axis=1, keepdims=True)
    hist = cnt / total
    h = jnp.clip(hist, 1e-6, 1.0 - 1e-6)
    h = jnp.log(h / (1.0 - h))
    h = lax.dot_general(h, w1_ref[...], (((1,), (1,)), ((), ())),
                        precision=lax.Precision.HIGHEST,
                        preferred_element_type=jnp.float32) \
        + jnp.reshape(b1_ref[...], (1, _BINS))
    h = jnp.where(h >= 0.0, h, 0.01 * h)
    d = lax.dot_general(h, w2_ref[...], (((1,), (1,)), ((), ())),
                        precision=lax.Precision.HIGHEST,
                        preferred_element_type=jnp.float32) \
        + jnp.reshape(b2_ref[...], (1, _BINS))
    mx = jnp.max(d, axis=1, keepdims=True)
    e = jnp.exp(d - mx)
    p = e / jnp.sum(e, axis=1, keepdims=True)
    rr = lax.broadcasted_iota(jnp.int32, (_BINS, _BINS), 0)
    cc = lax.broadcasted_iota(jnp.int32, (_BINS, _BINS), 1)
    tri = (rr <= cc).astype(jnp.float32)
    y = lax.dot_general(p, tri, (((1,), (0,)), ((), ())),
                        precision=lax.Precision.HIGHEST,
                        preferred_element_type=jnp.float32)
    e0 = y - p
    ji = lax.broadcasted_iota(jnp.int32, (1, _BINS), 1)
    j = ji.astype(jnp.float32)
    dxinv = jnp.where(ji == 0, 256.0, 128.0)
    x0 = jnp.where(ji == 0, 0.0, (2.0 * j - 1.0) / 256.0)
    bt = p * dxinv
    at = e0 - bt * x0
    zrows = jnp.zeros((_TROWS - _C, _BINS), jnp.float32)
    ta_ref[0:_C, :] = at
    ta_ref[_C:_TROWS, :] = zrows
    tb_ref[0:_C, :] = bt
    tb_ref[_C:_TROWS, :] = zrows


_fit_call = pl.pallas_call(
    _fit_kernel,
    out_shape=(
        jax.ShapeDtypeStruct((_TROWS, _BINS), jnp.float32),
        jax.ShapeDtypeStruct((_TROWS, _BINS), jnp.float32),
    ),
)


@functools.partial(
    pl.kernel,
    out_type=jax.ShapeDtypeStruct((_N,), jnp.float32),
    mesh=_MESH,
    compiler_params=pltpu.CompilerParams(needs_layout_passes=False),
    scratch_types=[
        pltpu.VMEM((_CHUNK,), jnp.float32),
        pltpu.VMEM((_CHUNK,), jnp.int32),
        pltpu.VMEM((_TABN,), jnp.float32),
        pltpu.VMEM((_TABN,), jnp.float32),
        pltpu.VMEM((_CHUNK,), jnp.float32),
    ],
)
def _interp_call(s_hbm, p_hbm, ta_hbm, tb_hbm, out_hbm, s_v, p_v, ta_v, tb_v, o_v):
    cid = lax.axis_index("c")
    sid = lax.axis_index("s")
    wid = cid * _NS + sid
    base = wid * _CHUNK
    pltpu.sync_copy(s_hbm.at[pl.ds(base, _CHUNK)], s_v)
    pltpu.sync_copy(p_hbm.at[pl.ds(base, _CHUNK)], p_v)
    pltpu.sync_copy(ta_hbm, ta_v)
    pltpu.sync_copy(tb_hbm, tb_v)

    tcoffs = _class_offsets(_BINS)
    ones = jnp.ones((_L,), jnp.float32)

    def mbody(o, carry):
        b0 = o * (_PERIOD * _L)
        for k in range(_PERIOD):
            off = b0 + k * _L
            s16 = s_v[pl.ds(off, _L)]
            p16 = p_v[pl.ds(off, _L)]
            iraw = (s16 * 128.0 + 0.5).astype(jnp.int32)
            idx = tcoffs[k] + jnp.minimum(iraw, _BINS - 1)
            a = plsc.load_gather(ta_v, [idx])
            b = plsc.load_gather(tb_v, [idx])
            w = a + b * s16
            y127 = a + b * (255.0 / 256.0)
            wedge = y127 + (1.0 - y127) * (256.0 * s16 - 255.0)
            w = jnp.where(iraw >= _BINS, wedge, w)
            o_v[pl.ds(off, _L)] = jnp.where(p16 == 1, w, ones)
        return carry

    lax.fori_loop(0, _OUTER, mbody, 0)
    pltpu.sync_copy(o_v, out_hbm.at[pl.ds(base, _CHUNK)])


def kernel(y_score, y_partial, W1, b1, W2, b2):
    s_flat = y_score.reshape(_N)
    p_flat = y_partial.astype(jnp.int32).reshape(_N)
    cnt = _hist_call(s_flat, p_flat)
    ta, tb = _fit_call(cnt, W1, b1, W2, b2)
    out = _interp_call(s_flat, p_flat, ta.reshape(_TABN), tb.reshape(_TABN))
    return out.reshape(_BATCH, _C)


# speedup vs baseline: 783.1391x; 1.1359x over previous
import functools

import jax
import jax.numpy as jnp
from jax import lax
from jax.experimental import pallas as pl
from jax.experimental.pallas import tpu as pltpu
from jax.experimental.pallas import tpu_sc as plsc

_BINS = 128
_C = 26
_BATCH = 16384
_NC, _NS, _L = 2, 16, 16
_NW = _NC * _NS
_ROWS = _BATCH // _NW
_PERIOD = 13
_FB = _C * _BINS
_PRIV = 8
_HSTRIDE = _FB + 1
_HWORDS = ((_PRIV * _HSTRIDE + 255) // 256) * 256
_BPS = _FB // _NS
_TROWS = 32
_TABN = _TROWS * _BINS

_RB_A = 256
_RB_C = 128

_MESH = plsc.VectorSubcoreMesh(core_axis_name="c", subcore_axis_name="s")


def _patterns():
    lane = lax.broadcasted_iota(jnp.int32, (_L,), 0)
    rincs, cidxs = [], []
    for k in range(_PERIOD):
        q, r = divmod(_L * k, _C)
        carry = (lane + r >= _C).astype(jnp.int32)
        rincs.append(carry + q)
        cidxs.append(lane + r - _C * carry)
    return rincs, cidxs


@functools.partial(
    pl.kernel,
    out_type=jax.ShapeDtypeStruct((_NC * _FB,), jnp.float32),
    mesh=_MESH,
    compiler_params=pltpu.CompilerParams(needs_layout_passes=False),
    scratch_types=[
        pltpu.VMEM((_RB_A, _C), jnp.float32),
        pltpu.VMEM((_RB_A, _C), jnp.int32),
        pltpu.VMEM((_HWORDS,), jnp.float32),
        pltpu.VMEM((_FB,), jnp.float32),
        pltpu.VMEM_SHARED((_NS * _FB,), jnp.float32),
        pltpu.VMEM((_NS * _BPS,), jnp.float32),
        pltpu.VMEM((_BPS,), jnp.float32),
    ],
)
def _hist_call(s_hbm, p_hbm, cnt_hbm, s_v, p_v, h_v, r_v, shared, cls_v, o_v):
    cid = lax.axis_index("c")
    sid = lax.axis_index("s")
    wid = cid * _NS + sid
    row0 = wid * _ROWS

    zero = jnp.zeros((_L,), jnp.float32)

    def zbody(i, carry):
        b = i * 256
        for k in range(16):
            h_v[pl.ds(b + k * _L, _L)] = zero
        return carry

    lax.fori_loop(0, _HWORDS // 256, zbody, 0)

    lane = lax.broadcasted_iota(jnp.int32, (_L,), 0)
    lane_off = (lane % _PRIV) * _HSTRIDE
    mlow = lane < _PRIV
    mhigh = jnp.logical_not(mlow)
    rincs, cidxs = _patterns()
    coffs = [c * _BINS + lane_off for c in cidxs]

    for blk in range(_ROWS // _RB_A):
        pltpu.sync_copy(s_hbm.at[pl.ds(row0 + blk * _RB_A, _RB_A)], s_v)
        pltpu.sync_copy(p_hbm.at[pl.ds(row0 + blk * _RB_A, _RB_A)], p_v)

        def mbody(o, carry):
            rbase = o * 8
            for k in range(_PERIOD):
                ridx = rbase + rincs[k]
                s16 = plsc.load_gather(s_v, [ridx, cidxs[k]])
                p16 = plsc.load_gather(p_v, [ridx, cidxs[k]])
                bin_ = jnp.minimum((s16 * 128.0).astype(jnp.int32), _BINS - 1)
                val = p16.astype(jnp.float32)
                idx = coffs[k] + bin_
                plsc.addupdate_scatter(h_v, [idx], val, mask=mlow)
                plsc.addupdate_scatter(h_v, [idx], val, mask=mhigh)
            return carry

        lax.fori_loop(0, _RB_A * _C // (_PERIOD * _L), mbody, 0)

    def rbody(j, carry):
        b = j * _L
        acc = h_v[pl.ds(b, _L)]
        for l in range(1, _PRIV):
            acc = acc + h_v[pl.ds(l * _HSTRIDE + b, _L)]
        r_v[pl.ds(b, _L)] = acc
        return carry

    lax.fori_loop(0, _FB // _L, rbody, 0)

    pltpu.sync_copy(r_v, shared.at[pl.ds(sid * _FB, _FB)])
    plsc.subcore_barrier()
    for l in range(_NS):
        pltpu.sync_copy(shared.at[pl.ds(l * _FB + sid * _BPS, _BPS)],
                        cls_v.at[pl.ds(l * _BPS, _BPS)])

    def cbody(k, carry):
        b = k * _L
        acc = cls_v[pl.ds(b, _L)]
        for l in range(1, _NS):
            acc = acc + cls_v[pl.ds(l * _BPS + b, _L)]
        o_v[pl.ds(b, _L)] = acc
        return carry

    lax.fori_loop(0, _BPS // _L, cbody, 0)
    pltpu.sync_copy(o_v, cnt_hbm.at[pl.ds(cid * _FB + sid * _BPS, _BPS)])


def _fit_kernel(cnt_ref, w1_ref, b1_ref, w2_ref, b2_ref, ta_ref, tb_ref):
    cnt2 = jnp.reshape(cnt_ref[...], (2 * _C, _BINS))
    cnt = cnt2[0:_C] + cnt2[_C:2 * _C]
    total = jnp.sum(cnt, axis=1, keepdims=True)
    hist = cnt / total
    h = jnp.clip(hist, 1e-6, 1.0 - 1e-6)
    h = jnp.log(h / (1.0 - h))
    h = lax.dot_general(h, w1_ref[...], (((1,), (1,)), ((), ())),
                        precision=lax.Precision.HIGHEST,
                        preferred_element_type=jnp.float32) \
        + jnp.reshape(b1_ref[...], (1, _BINS))
    h = jnp.where(h >= 0.0, h, 0.01 * h)
    d = lax.dot_general(h, w2_ref[...], (((1,), (1,)), ((), ())),
                        precision=lax.Precision.HIGHEST,
                        preferred_element_type=jnp.float32) \
        + jnp.reshape(b2_ref[...], (1, _BINS))
    mx = jnp.max(d, axis=1, keepdims=True)
    e = jnp.exp(d - mx)
    p = e / jnp.sum(e, axis=1, keepdims=True)
    rr = lax.broadcasted_iota(jnp.int32, (_BINS, _BINS), 0)
    cc = lax.broadcasted_iota(jnp.int32, (_BINS, _BINS), 1)
    tri = (rr <= cc).astype(jnp.float32)
    y = lax.dot_general(p, tri, (((1,), (0,)), ((), ())),
                        precision=lax.Precision.HIGHEST,
                        preferred_element_type=jnp.float32)
    e0 = y - p
    ji = lax.broadcasted_iota(jnp.int32, (1, _BINS), 1)
    j = ji.astype(jnp.float32)
    dxinv = jnp.where(ji == 0, 256.0, 128.0)
    x0 = jnp.where(ji == 0, 0.0, (2.0 * j - 1.0) / 256.0)
    bt = p * dxinv
    at = e0 - bt * x0
    zrows = jnp.zeros((_TROWS - _C, _BINS), jnp.float32)
    ta_ref[0:_C, :] = at
    ta_ref[_C:_TROWS, :] = zrows
    tb_ref[0:_C, :] = bt
    tb_ref[_C:_TROWS, :] = zrows


_fit_call = pl.pallas_call(
    _fit_kernel,
    out_shape=(
        jax.ShapeDtypeStruct((_TROWS, _BINS), jnp.float32),
        jax.ShapeDtypeStruct((_TROWS, _BINS), jnp.float32),
    ),
)


@functools.partial(
    pl.kernel,
    out_type=jax.ShapeDtypeStruct((_BATCH, _C), jnp.float32),
    mesh=_MESH,
    compiler_params=pltpu.CompilerParams(needs_layout_passes=False),
    scratch_types=[
        pltpu.VMEM((_RB_C, _C), jnp.float32),
        pltpu.VMEM((_RB_C, _C), jnp.int32),
        pltpu.VMEM((_TABN,), jnp.float32),
        pltpu.VMEM((_TABN,), jnp.float32),
        pltpu.VMEM((_RB_C, _C), jnp.float32),
    ],
)
def _interp_call(s_hbm, p_hbm, ta_hbm, tb_hbm, out_hbm, s_v, p_v, ta_v, tb_v, o_v):
    cid = lax.axis_index("c")
    sid = lax.axis_index("s")
    wid = cid * _NS + sid
    row0 = wid * _ROWS

    pltpu.sync_copy(ta_hbm, ta_v)
    pltpu.sync_copy(tb_hbm, tb_v)

    rincs, cidxs = _patterns()
    coffs = [c * _BINS for c in cidxs]
    ones = jnp.ones((_L,), jnp.float32)

    for blk in range(_ROWS // _RB_C):
        pltpu.sync_copy(s_hbm.at[pl.ds(row0 + blk * _RB_C, _RB_C)], s_v)
        pltpu.sync_copy(p_hbm.at[pl.ds(row0 + blk * _RB_C, _RB_C)], p_v)

        def mbody(o, carry):
            rbase = o * 8
            for k in range(_PERIOD):
                ridx = rbase + rincs[k]
                s16 = plsc.load_gather(s_v, [ridx, cidxs[k]])
                p16 = plsc.load_gather(p_v, [ridx, cidxs[k]])
                iraw = (s16 * 128.0 + 0.5).astype(jnp.int32)
                idx = coffs[k] + jnp.minimum(iraw, _BINS - 1)
                a = plsc.load_gather(ta_v, [idx])
                b = plsc.load_gather(tb_v, [idx])
                w = a + b * s16
                y127 = a + b * (255.0 / 256.0)
                wedge = y127 + (1.0 - y127) * (256.0 * s16 - 255.0)
                w = jnp.where(iraw >= _BINS, wedge, w)
                w = jnp.where(p16 == 1, w, ones)
                plsc.store_scatter(o_v, [ridx, cidxs[k]], w)
            return carry

        lax.fori_loop(0, _RB_C * _C // (_PERIOD * _L), mbody, 0)
        pltpu.sync_copy(o_v, out_hbm.at[pl.ds(row0 + blk * _RB_C, _RB_C)])


def kernel(y_score, y_partial, W1, b1, W2, b2):
    p2d = y_partial.astype(jnp.int32)
    cnt = _hist_call(y_score, p2d)
    ta, tb = _fit_call(cnt, W1, b1, W2, b2)
    return _interp_call(y_score, p2d, ta.reshape(_TABN), tb.reshape(_TABN))


# speedup vs baseline: 1239.4734x; 1.5827x over previous
import functools

import jax
import jax.numpy as jnp
from jax import lax
from jax.experimental import pallas as pl
from jax.experimental.pallas import tpu as pltpu
from jax.experimental.pallas import tpu_sc as plsc

_BINS = 128
_C = 26
_BATCH = 16384
_NC, _NS, _L = 2, 16, 16
_NW = _NC * _NS
_COLS = _BATCH // _NW
_CV = _COLS // _L
_FB = _C * _BINS
_PRIV = 8
_HSTRIDE = _FB + 1
_HWORDS = ((_PRIV * _HSTRIDE + 255) // 256) * 256
_BPS = _FB // _NS
_TROWS = 32
_TABN = _TROWS * _BINS

_MESH = plsc.VectorSubcoreMesh(core_axis_name="c", subcore_axis_name="s")


@functools.partial(
    pl.kernel,
    out_type=jax.ShapeDtypeStruct((_NC * _FB,), jnp.float32),
    mesh=_MESH,
    compiler_params=pltpu.CompilerParams(needs_layout_passes=False),
    scratch_types=[
        pltpu.VMEM((_C, _COLS), jnp.float32),
        pltpu.VMEM((_C, _COLS), jnp.int32),
        pltpu.VMEM((_HWORDS,), jnp.float32),
        pltpu.VMEM((_FB,), jnp.float32),
        pltpu.VMEM_SHARED((_NS * _FB,), jnp.float32),
        pltpu.VMEM((_NS * _BPS,), jnp.float32),
        pltpu.VMEM((_BPS,), jnp.float32),
    ],
)
def _hist_call(s_hbm, p_hbm, cnt_hbm, s_v, p_v, h_v, r_v, shared, cls_v, o_v):
    cid = lax.axis_index("c")
    sid = lax.axis_index("s")
    wid = cid * _NS + sid
    col0 = wid * _COLS
    pltpu.sync_copy(s_hbm.at[:, pl.ds(col0, _COLS)], s_v)
    pltpu.sync_copy(p_hbm.at[:, pl.ds(col0, _COLS)], p_v)

    zero = jnp.zeros((_L,), jnp.float32)

    def zbody(i, carry):
        b = i * 256
        for k in range(16):
            h_v[pl.ds(b + k * _L, _L)] = zero
        return carry

    lax.fori_loop(0, _HWORDS // 256, zbody, 0)

    lane = lax.broadcasted_iota(jnp.int32, (_L,), 0)
    lane_off = (lane % _PRIV) * _HSTRIDE
    mlow = lane < _PRIV
    mhigh = jnp.logical_not(mlow)

    def mbody(j, carry):
        b = j * _L
        for c in range(_C):
            s16 = s_v[c, pl.ds(b, _L)]
            p16 = p_v[c, pl.ds(b, _L)]
            bin_ = jnp.minimum((s16 * 128.0).astype(jnp.int32), _BINS - 1)
            idx = lane_off + (bin_ + c * _BINS)
            val = p16.astype(jnp.float32)
            plsc.addupdate_scatter(h_v, [idx], val, mask=mlow)
            plsc.addupdate_scatter(h_v, [idx], val, mask=mhigh)
        return carry

    lax.fori_loop(0, _CV, mbody, 0)

    def rbody(j, carry):
        b = j * _L
        acc = h_v[pl.ds(b, _L)]
        for l in range(1, _PRIV):
            acc = acc + h_v[pl.ds(l * _HSTRIDE + b, _L)]
        r_v[pl.ds(b, _L)] = acc
        return carry

    lax.fori_loop(0, _FB // _L, rbody, 0)

    pltpu.sync_copy(r_v, shared.at[pl.ds(sid * _FB, _FB)])
    plsc.subcore_barrier()
    for l in range(_NS):
        pltpu.sync_copy(shared.at[pl.ds(l * _FB + sid * _BPS, _BPS)],
                        cls_v.at[pl.ds(l * _BPS, _BPS)])

    def cbody(k, carry):
        b = k * _L
        acc = cls_v[pl.ds(b, _L)]
        for l in range(1, _NS):
            acc = acc + cls_v[pl.ds(l * _BPS + b, _L)]
        o_v[pl.ds(b, _L)] = acc
        return carry

    lax.fori_loop(0, _BPS // _L, cbody, 0)
    pltpu.sync_copy(o_v, cnt_hbm.at[pl.ds(cid * _FB + sid * _BPS, _BPS)])


def _fit_kernel(cnt_ref, w1_ref, b1_ref, w2_ref, b2_ref, ta_ref, tb_ref):
    cnt2 = jnp.reshape(cnt_ref[...], (2 * _C, _BINS))
    cnt = cnt2[0:_C] + cnt2[_C:2 * _C]
    total = jnp.sum(cnt, axis=1, keepdims=True)
    hist = cnt / total
    h = jnp.clip(hist, 1e-6, 1.0 - 1e-6)
    h = jnp.log(h / (1.0 - h))
    h = lax.dot_general(h, w1_ref[...], (((1,), (1,)), ((), ())),
                        precision=lax.Precision.HIGHEST,
                        preferred_element_type=jnp.float32) \
        + jnp.reshape(b1_ref[...], (1, _BINS))
    h = jnp.where(h >= 0.0, h, 0.01 * h)
    d = lax.dot_general(h, w2_ref[...], (((1,), (1,)), ((), ())),
                        precision=lax.Precision.HIGHEST,
                        preferred_element_type=jnp.float32) \
        + jnp.reshape(b2_ref[...], (1, _BINS))
    mx = jnp.max(d, axis=1, keepdims=True)
    e = jnp.exp(d - mx)
    p = e / jnp.sum(e, axis=1, keepdims=True)
    rr = lax.broadcasted_iota(jnp.int32, (_BINS, _BINS), 0)
    cc = lax.broadcasted_iota(jnp.int32, (_BINS, _BINS), 1)
    tri = (rr <= cc).astype(jnp.float32)
    y = lax.dot_general(p, tri, (((1,), (0,)), ((), ())),
                        precision=lax.Precision.HIGHEST,
                        preferred_element_type=jnp.float32)
    e0 = y - p
    ji = lax.broadcasted_iota(jnp.int32, (1, _BINS), 1)
    j = ji.astype(jnp.float32)
    dxinv = jnp.where(ji == 0, 256.0, 128.0)
    x0 = jnp.where(ji == 0, 0.0, (2.0 * j - 1.0) / 256.0)
    bt = p * dxinv
    at = e0 - bt * x0
    zrows = jnp.zeros((_TROWS - _C, _BINS), jnp.float32)
    ta_ref[0:_C, :] = at
    ta_ref[_C:_TROWS, :] = zrows
    tb_ref[0:_C, :] = bt
    tb_ref[_C:_TROWS, :] = zrows


_fit_call = pl.pallas_call(
    _fit_kernel,
    out_shape=(
        jax.ShapeDtypeStruct((_TROWS, _BINS), jnp.float32),
        jax.ShapeDtypeStruct((_TROWS, _BINS), jnp.float32),
    ),
)


@functools.partial(
    pl.kernel,
    out_type=jax.ShapeDtypeStruct((_C, _BATCH), jnp.float32),
    mesh=_MESH,
    compiler_params=pltpu.CompilerParams(needs_layout_passes=False),
    scratch_types=[
        pltpu.VMEM((_C, _COLS), jnp.float32),
        pltpu.VMEM((_C, _COLS), jnp.int32),
        pltpu.VMEM((_TABN,), jnp.float32),
        pltpu.VMEM((_TABN,), jnp.float32),
        pltpu.VMEM((_C, _COLS), jnp.float32),
    ],
)
def _interp_call(s_hbm, p_hbm, ta_hbm, tb_hbm, out_hbm, s_v, p_v, ta_v, tb_v, o_v):
    cid = lax.axis_index("c")
    sid = lax.axis_index("s")
    wid = cid * _NS + sid
    col0 = wid * _COLS

    pltpu.sync_copy(ta_hbm, ta_v)
    pltpu.sync_copy(tb_hbm, tb_v)
    pltpu.sync_copy(s_hbm.at[:, pl.ds(col0, _COLS)], s_v)
    pltpu.sync_copy(p_hbm.at[:, pl.ds(col0, _COLS)], p_v)

    ones = jnp.ones((_L,), jnp.float32)

    def mbody(j, carry):
        b = j * _L
        for c in range(_C):
            s16 = s_v[c, pl.ds(b, _L)]
            p16 = p_v[c, pl.ds(b, _L)]
            iraw = (s16 * 128.0 + 0.5).astype(jnp.int32)
            idx = jnp.minimum(iraw, _BINS - 1) + c * _BINS
            a = plsc.load_gather(ta_v, [idx])
            bb = plsc.load_gather(tb_v, [idx])
            w = a + bb * s16
            y127 = a + bb * (255.0 / 256.0)
            wedge = y127 + (1.0 - y127) * (256.0 * s16 - 255.0)
            w = jnp.where(iraw >= _BINS, wedge, w)
            o_v[c, pl.ds(b, _L)] = jnp.where(p16 == 1, w, ones)
        return carry

    lax.fori_loop(0, _CV, mbody, 0)
    pltpu.sync_copy(o_v, out_hbm.at[:, pl.ds(col0, _COLS)])


def kernel(y_score, y_partial, W1, b1, W2, b2):
    s_t = y_score.T
    p_t = y_partial.astype(jnp.int32).T
    cnt = _hist_call(s_t, p_t)
    ta, tb = _fit_call(cnt, W1, b1, W2, b2)
    out_t = _interp_call(s_t, p_t, ta.reshape(_TABN), tb.reshape(_TABN))
    return out_t.T


# speedup vs baseline: 1494.1114x; 1.2054x over previous
import functools

import jax
import jax.numpy as jnp
from jax import lax
from jax.experimental import pallas as pl
from jax.experimental.pallas import tpu as pltpu
from jax.experimental.pallas import tpu_sc as plsc

_BINS = 128
_C = 26
_BATCH = 16384
_NC, _NS, _L = 2, 16, 16
_NW = _NC * _NS
_COLS = _BATCH // _NW
_CV = _COLS // _L
_FB = _C * _BINS
_PRIV = 8
_HSTRIDE = _FB + 1
_HWORDS = ((_PRIV * _HSTRIDE + 255) // 256) * 256
_BPS = _FB // _NS
_TROWS = 32
_TABN = _TROWS * _BINS

_MESH = plsc.VectorSubcoreMesh(core_axis_name="c", subcore_axis_name="s")


@functools.partial(
    pl.kernel,
    out_type=jax.ShapeDtypeStruct((_NC * _FB,), jnp.float32),
    mesh=_MESH,
    compiler_params=pltpu.CompilerParams(needs_layout_passes=False),
    scratch_types=[
        pltpu.VMEM((_C, _COLS), jnp.float32),
        pltpu.VMEM((_C, _COLS), jnp.int32),
        pltpu.VMEM((_HWORDS,), jnp.float32),
        pltpu.VMEM((_FB,), jnp.float32),
        pltpu.VMEM_SHARED((_NS * _FB,), jnp.float32),
        pltpu.VMEM((_NS * _BPS,), jnp.float32),
        pltpu.VMEM((_BPS,), jnp.float32),
    ],
)
def _hist_call(s_hbm, p_hbm, cnt_hbm, s_v, p_v, h_v, r_v, shared, cls_v, o_v):
    cid = lax.axis_index("c")
    sid = lax.axis_index("s")
    wid = cid * _NS + sid
    col0 = wid * _COLS
    pltpu.sync_copy(s_hbm.at[:, pl.ds(col0, _COLS)], s_v)
    pltpu.sync_copy(p_hbm.at[:, pl.ds(col0, _COLS)], p_v)

    zero = jnp.zeros((_L,), jnp.float32)

    @plsc.parallel_loop(0, _HWORDS // 256, 1, unroll=2)
    def zbody(i):
        b = i * 256
        for k in range(16):
            h_v[pl.ds(b + k * _L, _L)] = zero

    lane = lax.broadcasted_iota(jnp.int32, (_L,), 0)
    lane_off = (lane % _PRIV) * _HSTRIDE
    mlow = lane < _PRIV
    mhigh = jnp.logical_not(mlow)

    @plsc.parallel_loop(0, _CV, 1, unroll=2)
    def mbody(j):
        b = j * _L
        for c in range(_C):
            s16 = s_v[c, pl.ds(b, _L)]
            p16 = p_v[c, pl.ds(b, _L)]
            bin_ = jnp.minimum((s16 * 128.0).astype(jnp.int32), _BINS - 1)
            idx = lane_off + (bin_ + c * _BINS)
            val = p16.astype(jnp.float32)
            plsc.addupdate_scatter(h_v, [idx], val, mask=mlow)
            plsc.addupdate_scatter(h_v, [idx], val, mask=mhigh)

    @plsc.parallel_loop(0, _FB // _L, 1, unroll=4)
    def rbody(j):
        b = j * _L
        acc = h_v[pl.ds(b, _L)]
        for l in range(1, _PRIV):
            acc = acc + h_v[pl.ds(l * _HSTRIDE + b, _L)]
        r_v[pl.ds(b, _L)] = acc

    pltpu.sync_copy(r_v, shared.at[pl.ds(sid * _FB, _FB)])
    plsc.subcore_barrier()
    for l in range(_NS):
        pltpu.sync_copy(shared.at[pl.ds(l * _FB + sid * _BPS, _BPS)],
                        cls_v.at[pl.ds(l * _BPS, _BPS)])

    @plsc.parallel_loop(0, _BPS // _L, 1, unroll=2)
    def cbody(k):
        b = k * _L
        acc = cls_v[pl.ds(b, _L)]
        for l in range(1, _NS):
            acc = acc + cls_v[pl.ds(l * _BPS + b, _L)]
        o_v[pl.ds(b, _L)] = acc
    pltpu.sync_copy(o_v, cnt_hbm.at[pl.ds(cid * _FB + sid * _BPS, _BPS)])


def _fit_kernel(cnt_ref, w1_ref, b1_ref, w2_ref, b2_ref, ta_ref, tb_ref):
    cnt2 = jnp.reshape(cnt_ref[...], (2 * _C, _BINS))
    cnt = cnt2[0:_C] + cnt2[_C:2 * _C]
    total = jnp.sum(cnt, axis=1, keepdims=True)
    hist = cnt / total
    h = jnp.clip(hist, 1e-6, 1.0 - 1e-6)
    h = jnp.log(h / (1.0 - h))
    h = lax.dot_general(h, w1_ref[...], (((1,), (1,)), ((), ())),
                        precision=lax.Precision.HIGHEST,
                        preferred_element_type=jnp.float32) \
        + jnp.reshape(b1_ref[...], (1, _BINS))
    h = jnp.where(h >= 0.0, h, 0.01 * h)
    d = lax.dot_general(h, w2_ref[...], (((1,), (1,)), ((), ())),
                        precision=lax.Precision.HIGHEST,
                        preferred_element_type=jnp.float32) \
        + jnp.reshape(b2_ref[...], (1, _BINS))
    mx = jnp.max(d, axis=1, keepdims=True)
    e = jnp.exp(d - mx)
    p = e / jnp.sum(e, axis=1, keepdims=True)
    rr = lax.broadcasted_iota(jnp.int32, (_BINS, _BINS), 0)
    cc = lax.broadcasted_iota(jnp.int32, (_BINS, _BINS), 1)
    tri = (rr <= cc).astype(jnp.float32)
    y = lax.dot_general(p, tri, (((1,), (0,)), ((), ())),
                        precision=lax.Precision.HIGHEST,
                        preferred_element_type=jnp.float32)
    e0 = y - p
    ji = lax.broadcasted_iota(jnp.int32, (1, _BINS), 1)
    j = ji.astype(jnp.float32)
    dxinv = jnp.where(ji == 0, 256.0, 128.0)
    x0 = jnp.where(ji == 0, 0.0, (2.0 * j - 1.0) / 256.0)
    bt = p * dxinv
    at = e0 - bt * x0
    zrows = jnp.zeros((_TROWS - _C, _BINS), jnp.float32)
    ta_ref[0:_C, :] = at
    ta_ref[_C:_TROWS, :] = zrows
    tb_ref[0:_C, :] = bt
    tb_ref[_C:_TROWS, :] = zrows


_fit_call = pl.pallas_call(
    _fit_kernel,
    out_shape=(
        jax.ShapeDtypeStruct((_TROWS, _BINS), jnp.float32),
        jax.ShapeDtypeStruct((_TROWS, _BINS), jnp.float32),
    ),
)


@functools.partial(
    pl.kernel,
    out_type=jax.ShapeDtypeStruct((_C, _BATCH), jnp.float32),
    mesh=_MESH,
    compiler_params=pltpu.CompilerParams(needs_layout_passes=False),
    scratch_types=[
        pltpu.VMEM((_C, _COLS), jnp.float32),
        pltpu.VMEM((_C, _COLS), jnp.int32),
        pltpu.VMEM((_TABN,), jnp.float32),
        pltpu.VMEM((_TABN,), jnp.float32),
        pltpu.VMEM((_C, _COLS), jnp.float32),
    ],
)
def _interp_call(s_hbm, p_hbm, ta_hbm, tb_hbm, out_hbm, s_v, p_v, ta_v, tb_v, o_v):
    cid = lax.axis_index("c")
    sid = lax.axis_index("s")
    wid = cid * _NS + sid
    col0 = wid * _COLS

    pltpu.sync_copy(ta_hbm, ta_v)
    pltpu.sync_copy(tb_hbm, tb_v)
    pltpu.sync_copy(s_hbm.at[:, pl.ds(col0, _COLS)], s_v)
    pltpu.sync_copy(p_hbm.at[:, pl.ds(col0, _COLS)], p_v)

    ones = jnp.ones((_L,), jnp.float32)

    @plsc.parallel_loop(0, _CV, 1, unroll=2)
    def mbody(j):
        b = j * _L
        for c in range(_C):
            s16 = s_v[c, pl.ds(b, _L)]
            p16 = p_v[c, pl.ds(b, _L)]
            iraw = (s16 * 128.0 + 0.5).astype(jnp.int32)
            idx = jnp.minimum(iraw, _BINS - 1) + c * _BINS
            a = plsc.load_gather(ta_v, [idx])
            bb = plsc.load_gather(tb_v, [idx])
            w = a + bb * s16
            y127 = a + bb * (255.0 / 256.0)
            wedge = y127 + (1.0 - y127) * (256.0 * s16 - 255.0)
            w = jnp.where(iraw >= _BINS, wedge, w)
            o_v[c, pl.ds(b, _L)] = jnp.where(p16 == 1, w, ones)
    pltpu.sync_copy(o_v, out_hbm.at[:, pl.ds(col0, _COLS)])


def kernel(y_score, y_partial, W1, b1, W2, b2):
    s_t = y_score.T
    p_t = y_partial.astype(jnp.int32).T
    cnt = _hist_call(s_t, p_t)
    ta, tb = _fit_call(cnt, W1, b1, W2, b2)
    out_t = _interp_call(s_t, p_t, ta.reshape(_TABN), tb.reshape(_TABN))
    return out_t.T


# speedup vs baseline: 1715.9758x; 1.1485x over previous
import functools

import jax
import jax.numpy as jnp
from jax import lax
from jax.experimental import pallas as pl
from jax.experimental.pallas import tpu as pltpu
from jax.experimental.pallas import tpu_sc as plsc

_BINS = 128
_C = 26
_BATCH = 16384
_NC, _NS, _L = 2, 16, 16
_NW = _NC * _NS
_COLS = _BATCH // _NW
_CV = _COLS // _L
_FB = _C * _BINS
_PRIV = 8
_HSTRIDE = _FB + 1
_HWORDS = ((_PRIV * _HSTRIDE + 255) // 256) * 256
_BPS = _FB // _NS
_TROWS = 32
_TABN = _TROWS * _BINS

_MESH = plsc.VectorSubcoreMesh(core_axis_name="c", subcore_axis_name="s")


@functools.partial(
    pl.kernel,
    out_type=jax.ShapeDtypeStruct((_NC * _FB,), jnp.float32),
    mesh=_MESH,
    compiler_params=pltpu.CompilerParams(needs_layout_passes=False),
    scratch_types=[
        pltpu.VMEM((_C, _COLS), jnp.float32),
        pltpu.VMEM((_C, _COLS), jnp.int32),
        pltpu.VMEM((_HWORDS,), jnp.float32),
        pltpu.VMEM((_FB,), jnp.float32),
        pltpu.VMEM_SHARED((_NS * _FB,), jnp.float32),
        pltpu.VMEM((_NS * _BPS,), jnp.float32),
        pltpu.VMEM((_BPS,), jnp.float32),
        pltpu.SemaphoreType.DMA,
    ],
)
def _hist_call(s_hbm, p_hbm, cnt_hbm, s_v, p_v, h_v, r_v, shared, cls_v, o_v, sem):
    cid = lax.axis_index("c")
    sid = lax.axis_index("s")
    wid = cid * _NS + sid
    col0 = wid * _COLS
    h_s = pltpu.async_copy(s_hbm.at[:, pl.ds(col0, _COLS)], s_v, sem)
    h_p = pltpu.async_copy(p_hbm.at[:, pl.ds(col0, _COLS)], p_v, sem)

    zero = jnp.zeros((_L,), jnp.float32)

    @plsc.parallel_loop(0, _HWORDS // 256, 1, unroll=2)
    def zbody(i):
        b = i * 256
        for k in range(16):
            h_v[pl.ds(b + k * _L, _L)] = zero

    h_s.wait()
    h_p.wait()

    lane = lax.broadcasted_iota(jnp.int32, (_L,), 0)
    lane_off = (lane % _PRIV) * _HSTRIDE
    mlow = lane < _PRIV
    mhigh = jnp.logical_not(mlow)

    @plsc.parallel_loop(0, _CV, 1, unroll=2)
    def mbody(j):
        b = j * _L
        for c in range(_C):
            s16 = s_v[c, pl.ds(b, _L)]
            p16 = p_v[c, pl.ds(b, _L)]
            bin_ = jnp.minimum((s16 * 128.0).astype(jnp.int32), _BINS - 1)
            idx = lane_off + (bin_ + c * _BINS)
            val = p16.astype(jnp.float32)
            plsc.addupdate_scatter(h_v, [idx], val, mask=mlow)
            plsc.addupdate_scatter(h_v, [idx], val, mask=mhigh)

    @plsc.parallel_loop(0, _FB // _L, 1, unroll=4)
    def rbody(j):
        b = j * _L
        acc = h_v[pl.ds(b, _L)]
        for l in range(1, _PRIV):
            acc = acc + h_v[pl.ds(l * _HSTRIDE + b, _L)]
        r_v[pl.ds(b, _L)] = acc

    pltpu.sync_copy(r_v, shared.at[pl.ds(sid * _FB, _FB)])
    plsc.subcore_barrier()
    handles = [
        pltpu.async_copy(shared.at[pl.ds(l * _FB + sid * _BPS, _BPS)],
                         cls_v.at[pl.ds(l * _BPS, _BPS)], sem)
        for l in range(_NS)
    ]
    for h in handles:
        h.wait()

    @plsc.parallel_loop(0, _BPS // _L, 1, unroll=2)
    def cbody(k):
        b = k * _L
        acc = cls_v[pl.ds(b, _L)]
        for l in range(1, _NS):
            acc = acc + cls_v[pl.ds(l * _BPS + b, _L)]
        o_v[pl.ds(b, _L)] = acc
    pltpu.sync_copy(o_v, cnt_hbm.at[pl.ds(cid * _FB + sid * _BPS, _BPS)])


def _fit_kernel(cnt_ref, w1_ref, b1_ref, w2_ref, b2_ref, ta_ref, tb_ref):
    cnt2 = jnp.reshape(cnt_ref[...], (2 * _C, _BINS))
    cnt = cnt2[0:_C] + cnt2[_C:2 * _C]
    total = jnp.sum(cnt, axis=1, keepdims=True)
    hist = cnt / total
    h = jnp.clip(hist, 1e-6, 1.0 - 1e-6)
    h = jnp.log(h / (1.0 - h))
    h = lax.dot_general(h, w1_ref[...], (((1,), (1,)), ((), ())),
                        precision=lax.Precision.HIGHEST,
                        preferred_element_type=jnp.float32) \
        + jnp.reshape(b1_ref[...], (1, _BINS))
    h = jnp.where(h >= 0.0, h, 0.01 * h)
    d = lax.dot_general(h, w2_ref[...], (((1,), (1,)), ((), ())),
                        precision=lax.Precision.HIGHEST,
                        preferred_element_type=jnp.float32) \
        + jnp.reshape(b2_ref[...], (1, _BINS))
    mx = jnp.max(d, axis=1, keepdims=True)
    e = jnp.exp(d - mx)
    p = e / jnp.sum(e, axis=1, keepdims=True)
    rr = lax.broadcasted_iota(jnp.int32, (_BINS, _BINS), 0)
    cc = lax.broadcasted_iota(jnp.int32, (_BINS, _BINS), 1)
    tri = (rr <= cc).astype(jnp.float32)
    y = lax.dot_general(p, tri, (((1,), (0,)), ((), ())),
                        precision=lax.Precision.HIGHEST,
                        preferred_element_type=jnp.float32)
    e0 = y - p
    ji = lax.broadcasted_iota(jnp.int32, (1, _BINS), 1)
    j = ji.astype(jnp.float32)
    dxinv = jnp.where(ji == 0, 256.0, 128.0)
    x0 = jnp.where(ji == 0, 0.0, (2.0 * j - 1.0) / 256.0)
    bt = p * dxinv
    at = e0 - bt * x0
    zrows = jnp.zeros((_TROWS - _C, _BINS), jnp.float32)
    ta_ref[0:_C, :] = at
    ta_ref[_C:_TROWS, :] = zrows
    tb_ref[0:_C, :] = bt
    tb_ref[_C:_TROWS, :] = zrows


_fit_call = pl.pallas_call(
    _fit_kernel,
    out_shape=(
        jax.ShapeDtypeStruct((_TROWS, _BINS), jnp.float32),
        jax.ShapeDtypeStruct((_TROWS, _BINS), jnp.float32),
    ),
)


@functools.partial(
    pl.kernel,
    out_type=jax.ShapeDtypeStruct((_C, _BATCH), jnp.float32),
    mesh=_MESH,
    compiler_params=pltpu.CompilerParams(needs_layout_passes=False),
    scratch_types=[
        pltpu.VMEM((_C, _COLS), jnp.float32),
        pltpu.VMEM((_C, _COLS), jnp.int32),
        pltpu.VMEM((_TABN,), jnp.float32),
        pltpu.VMEM((_TABN,), jnp.float32),
        pltpu.VMEM((_C, _COLS), jnp.float32),
        pltpu.SemaphoreType.DMA,
    ],
)
def _interp_call(s_hbm, p_hbm, ta_hbm, tb_hbm, out_hbm,
                 s_v, p_v, ta_v, tb_v, o_v, sem):
    cid = lax.axis_index("c")
    sid = lax.axis_index("s")
    wid = cid * _NS + sid
    col0 = wid * _COLS

    handles = [
        pltpu.async_copy(ta_hbm, ta_v, sem),
        pltpu.async_copy(tb_hbm, tb_v, sem),
        pltpu.async_copy(s_hbm.at[:, pl.ds(col0, _COLS)], s_v, sem),
        pltpu.async_copy(p_hbm.at[:, pl.ds(col0, _COLS)], p_v, sem),
    ]
    for h in handles:
        h.wait()

    ones = jnp.ones((_L,), jnp.float32)

    @plsc.parallel_loop(0, _CV, 1, unroll=4)
    def mbody(j):
        b = j * _L
        for c in range(_C):
            s16 = s_v[c, pl.ds(b, _L)]
            p16 = p_v[c, pl.ds(b, _L)]
            iraw = (s16 * 128.0 + 0.5).astype(jnp.int32)
            idx = jnp.minimum(iraw, _BINS - 1) + c * _BINS
            a = plsc.load_gather(ta_v, [idx])
            bb = plsc.load_gather(tb_v, [idx])
            w = a + bb * s16
            y127 = a + bb * (255.0 / 256.0)
            wedge = y127 + (1.0 - y127) * (256.0 * s16 - 255.0)
            w = jnp.where(iraw >= _BINS, wedge, w)
            o_v[c, pl.ds(b, _L)] = jnp.where(p16 == 1, w, ones)
    pltpu.sync_copy(o_v, out_hbm.at[:, pl.ds(col0, _COLS)])


def kernel(y_score, y_partial, W1, b1, W2, b2):
    s_t = y_score.T
    p_t = y_partial.astype(jnp.int32).T
    cnt = _hist_call(s_t, p_t)
    ta, tb = _fit_call(cnt, W1, b1, W2, b2)
    out_t = _interp_call(s_t, p_t, ta.reshape(_TABN), tb.reshape(_TABN))
    return out_t.T


# speedup vs baseline: 1846.5717x; 1.0761x over previous
import functools

import jax
import jax.numpy as jnp
from jax import lax
from jax.experimental import pallas as pl
from jax.experimental.pallas import tpu as pltpu
from jax.experimental.pallas import tpu_sc as plsc

_BINS = 128
_C = 26
_BATCH = 16384
_NC, _NS, _L = 2, 16, 16
_NW = _NC * _NS
_COLS = _BATCH // _NW
_CV = _COLS // _L
_FB = _C * _BINS
_PRIV = 8
_HSTRIDE = _FB + 1
_HWORDS = ((_PRIV * _HSTRIDE + 255) // 256) * 256
_BPS = _FB // _NS
_TROWS = 32
_TABN = _TROWS * _BINS

_MESH = plsc.VectorSubcoreMesh(core_axis_name="c", subcore_axis_name="s")


@functools.partial(
    pl.kernel,
    out_type=jax.ShapeDtypeStruct((_NC * _FB,), jnp.float32),
    mesh=_MESH,
    compiler_params=pltpu.CompilerParams(needs_layout_passes=False),
    scratch_types=[
        pltpu.VMEM((_C, _COLS), jnp.float32),
        pltpu.VMEM((_C, _COLS), jnp.int32),
        pltpu.VMEM((_HWORDS,), jnp.float32),
        pltpu.VMEM((_FB,), jnp.float32),
        pltpu.VMEM_SHARED((_NS * _FB,), jnp.float32),
        pltpu.VMEM((_NS * _BPS,), jnp.float32),
        pltpu.VMEM((_BPS,), jnp.float32),
        pltpu.SemaphoreType.DMA,
    ],
)
def _hist_call(s_hbm, p_hbm, cnt_hbm, s_v, p_v, h_v, r_v, shared, cls_v, o_v, sem):
    cid = lax.axis_index("c")
    sid = lax.axis_index("s")
    wid = cid * _NS + sid
    col0 = wid * _COLS
    h_s = pltpu.async_copy(s_hbm.at[:, pl.ds(col0, _COLS)], s_v, sem)
    h_p = pltpu.async_copy(p_hbm.at[:, pl.ds(col0, _COLS)], p_v, sem)

    zero = jnp.zeros((_L,), jnp.float32)

    @plsc.parallel_loop(0, _HWORDS // 256, 1, unroll=2)
    def zbody(i):
        b = i * 256
        for k in range(16):
            h_v[pl.ds(b + k * _L, _L)] = zero

    h_s.wait()
    h_p.wait()

    lane = lax.broadcasted_iota(jnp.int32, (_L,), 0)
    lane_off = (lane % _PRIV) * _HSTRIDE
    mlow = lane < _PRIV
    mhigh = jnp.logical_not(mlow)

    @plsc.parallel_loop(0, _CV, 1, unroll=2)
    def mbody(j):
        b = j * _L
        for c0 in range(0, _C, 4):
            cg = range(c0, min(c0 + 4, _C))
            ss = [s_v[c, pl.ds(b, _L)] for c in cg]
            pp = [p_v[c, pl.ds(b, _L)] for c in cg]
            idxs = [lane_off +
                    (jnp.minimum((s * 128.0).astype(jnp.int32), _BINS - 1)
                     + c * _BINS)
                    for c, s in zip(cg, ss)]
            vals = [p.astype(jnp.float32) for p in pp]
            for idx, val in zip(idxs, vals):
                plsc.addupdate_scatter(h_v, [idx], val, mask=mlow)
                plsc.addupdate_scatter(h_v, [idx], val, mask=mhigh)

    @plsc.parallel_loop(0, _FB // _L, 1, unroll=4)
    def rbody(j):
        b = j * _L
        acc = h_v[pl.ds(b, _L)]
        for l in range(1, _PRIV):
            acc = acc + h_v[pl.ds(l * _HSTRIDE + b, _L)]
        r_v[pl.ds(b, _L)] = acc

    pltpu.sync_copy(r_v, shared.at[pl.ds(sid * _FB, _FB)])
    plsc.subcore_barrier()
    handles = [
        pltpu.async_copy(shared.at[pl.ds(l * _FB + sid * _BPS, _BPS)],
                         cls_v.at[pl.ds(l * _BPS, _BPS)], sem)
        for l in range(_NS)
    ]
    for h in handles:
        h.wait()

    @plsc.parallel_loop(0, _BPS // _L, 1, unroll=2)
    def cbody(k):
        b = k * _L
        acc = cls_v[pl.ds(b, _L)]
        for l in range(1, _NS):
            acc = acc + cls_v[pl.ds(l * _BPS + b, _L)]
        o_v[pl.ds(b, _L)] = acc
    pltpu.sync_copy(o_v, cnt_hbm.at[pl.ds(cid * _FB + sid * _BPS, _BPS)])


def _fit_kernel(cnt_ref, w1_ref, b1_ref, w2_ref, b2_ref, ta_ref, tb_ref):
    cnt2 = jnp.reshape(cnt_ref[...], (2 * _C, _BINS))
    cnt = cnt2[0:_C] + cnt2[_C:2 * _C]
    total = jnp.sum(cnt, axis=1, keepdims=True)
    hist = cnt / total
    h = jnp.clip(hist, 1e-6, 1.0 - 1e-6)
    h = jnp.log(h / (1.0 - h))
    h = lax.dot_general(h, w1_ref[...], (((1,), (1,)), ((), ())),
                        precision=lax.Precision.HIGHEST,
                        preferred_element_type=jnp.float32) \
        + jnp.reshape(b1_ref[...], (1, _BINS))
    h = jnp.where(h >= 0.0, h, 0.01 * h)
    d = lax.dot_general(h, w2_ref[...], (((1,), (1,)), ((), ())),
                        precision=lax.Precision.HIGHEST,
                        preferred_element_type=jnp.float32) \
        + jnp.reshape(b2_ref[...], (1, _BINS))
    mx = jnp.max(d, axis=1, keepdims=True)
    e = jnp.exp(d - mx)
    p = e / jnp.sum(e, axis=1, keepdims=True)
    rr = lax.broadcasted_iota(jnp.int32, (_BINS, _BINS), 0)
    cc = lax.broadcasted_iota(jnp.int32, (_BINS, _BINS), 1)
    tri = (rr <= cc).astype(jnp.float32)
    y = lax.dot_general(p, tri, (((1,), (0,)), ((), ())),
                        precision=lax.Precision.HIGHEST,
                        preferred_element_type=jnp.float32)
    e0 = y - p
    ji = lax.broadcasted_iota(jnp.int32, (1, _BINS), 1)
    j = ji.astype(jnp.float32)
    dxinv = jnp.where(ji == 0, 256.0, 128.0)
    x0 = jnp.where(ji == 0, 0.0, (2.0 * j - 1.0) / 256.0)
    bt = p * dxinv
    at = e0 - bt * x0
    zrows = jnp.zeros((_TROWS - _C, _BINS), jnp.float32)
    ta_ref[0:_C, :] = at
    ta_ref[_C:_TROWS, :] = zrows
    tb_ref[0:_C, :] = bt
    tb_ref[_C:_TROWS, :] = zrows


_fit_call = pl.pallas_call(
    _fit_kernel,
    out_shape=(
        jax.ShapeDtypeStruct((_TROWS, _BINS), jnp.float32),
        jax.ShapeDtypeStruct((_TROWS, _BINS), jnp.float32),
    ),
)


@functools.partial(
    pl.kernel,
    out_type=jax.ShapeDtypeStruct((_C, _BATCH), jnp.float32),
    mesh=_MESH,
    compiler_params=pltpu.CompilerParams(needs_layout_passes=False),
    scratch_types=[
        pltpu.VMEM((_C, _COLS), jnp.float32),
        pltpu.VMEM((_C, _COLS), jnp.int32),
        pltpu.VMEM((_TABN,), jnp.float32),
        pltpu.VMEM((_TABN,), jnp.float32),
        pltpu.VMEM((_C, _COLS), jnp.float32),
        pltpu.SemaphoreType.DMA,
    ],
)
def _interp_call(s_hbm, p_hbm, ta_hbm, tb_hbm, out_hbm,
                 s_v, p_v, ta_v, tb_v, o_v, sem):
    cid = lax.axis_index("c")
    sid = lax.axis_index("s")
    wid = cid * _NS + sid
    col0 = wid * _COLS

    handles = [
        pltpu.async_copy(ta_hbm, ta_v, sem),
        pltpu.async_copy(tb_hbm, tb_v, sem),
        pltpu.async_copy(s_hbm.at[:, pl.ds(col0, _COLS)], s_v, sem),
        pltpu.async_copy(p_hbm.at[:, pl.ds(col0, _COLS)], p_v, sem),
    ]
    for h in handles:
        h.wait()

    ones = jnp.ones((_L,), jnp.float32)

    @plsc.parallel_loop(0, _CV, 1, unroll=2)
    def mbody(j):
        b = j * _L
        for c0 in range(0, _C, 4):
            cg = range(c0, min(c0 + 4, _C))
            ss = [s_v[c, pl.ds(b, _L)] for c in cg]
            pp = [p_v[c, pl.ds(b, _L)] for c in cg]
            iraws = [(s * 128.0 + 0.5).astype(jnp.int32) for s in ss]
            idxs = [jnp.minimum(ir, _BINS - 1) + c * _BINS
                    for c, ir in zip(cg, iraws)]
            aa = [plsc.load_gather(ta_v, [idx]) for idx in idxs]
            bbs = [plsc.load_gather(tb_v, [idx]) for idx in idxs]
            for c, s16, p16, iraw, a, bb in zip(cg, ss, pp, iraws, aa, bbs):
                w = a + bb * s16
                y127 = a + bb * (255.0 / 256.0)
                wedge = y127 + (1.0 - y127) * (256.0 * s16 - 255.0)
                w = jnp.where(iraw >= _BINS, wedge, w)
                o_v[c, pl.ds(b, _L)] = jnp.where(p16 == 1, w, ones)
    pltpu.sync_copy(o_v, out_hbm.at[:, pl.ds(col0, _COLS)])


def kernel(y_score, y_partial, W1, b1, W2, b2):
    s_t = y_score.T
    p_t = y_partial.astype(jnp.int32).T
    cnt = _hist_call(s_t, p_t)
    ta, tb = _fit_call(cnt, W1, b1, W2, b2)
    out_t = _interp_call(s_t, p_t, ta.reshape(_TABN), tb.reshape(_TABN))
    return out_t.T
